# Initial kernel scaffold; baseline (speedup 1.0000x reference)
import jax
import jax.numpy as jnp
from jax import lax
from jax.experimental import pallas as pl
from jax.experimental.pallas import tpu as pltpu
from jax.experimental.pallas import tpu_sc as plsc

N = 10000
D = 128
E = 320000

NC = 2
NS = 16
NW = NC * NS

C = 128
CPW = 79
EP = NW * CPW * C
NP = 10240
RPS = NP // NS

R = 1000
NBLK = N // R


def _make_sc_agg(d_out: int, with_deg: bool):
  mesh = plsc.VectorSubcoreMesh(
      core_axis_name="c", subcore_axis_name="s", num_cores=NC, num_subcores=NS)

  out_type = [jax.ShapeDtypeStruct((NP, d_out), jnp.float32),
              jax.ShapeDtypeStruct((NP, d_out), jnp.float32)]
  scratch = [
      pltpu.VMEM_SHARED((NP, d_out), jnp.float32),
      pltpu.VMEM((CPW, C), jnp.int32),
      pltpu.VMEM((CPW, C), jnp.int32),
      pltpu.VMEM((C, d_out), jnp.float32),
      pltpu.SemaphoreType.DMA,
  ]
  if with_deg:
    out_type += [jax.ShapeDtypeStruct((NP, 1), jnp.float32),
                 jax.ShapeDtypeStruct((NP, 1), jnp.float32)]
    scratch += [
        pltpu.VMEM_SHARED((NP, 1), jnp.float32),
        pltpu.VMEM((C, 1), jnp.float32),
    ]

  def body(p_hbm, src_hbm, dst_hbm, zrow_hbm, zone_hbm, one_hbm, *refs):
    if with_deg:
      agg_a, agg_b, deg_a, deg_b, acc, srcv, dstv, rows, sem, dacc, onesv = refs
    else:
      agg_a, agg_b, acc, srcv, dstv, rows, sem = refs
    cid = lax.axis_index("c")
    sid = lax.axis_index("s")
    wid = sid * NC + cid
    r0 = sid * RPS

    pltpu.sync_copy(src_hbm.at[pl.ds(wid * CPW, CPW)], srcv)
    pltpu.sync_copy(dst_hbm.at[pl.ds(wid * CPW, CPW)], dstv)
    pltpu.sync_copy(zrow_hbm, acc.at[pl.ds(r0, RPS)])
    if with_deg:
      pltpu.sync_copy(zone_hbm, dacc.at[pl.ds(r0, RPS)])
      pltpu.sync_copy(one_hbm, onesv)
    plsc.subcore_barrier()

    def step(j, carry):
      pltpu.async_copy(p_hbm.at[srcv.at[j]], rows, sem).wait()
      pltpu.sync_copy(rows, acc.at[dstv.at[j]], add=True)
      if with_deg:
        pltpu.sync_copy(onesv, dacc.at[dstv.at[j]], add=True)
      return carry

    lax.fori_loop(0, CPW, step, 0)
    plsc.subcore_barrier()

    @pl.when(cid == 0)
    def _():
      pltpu.sync_copy(acc.at[pl.ds(r0, RPS)], agg_a.at[pl.ds(r0, RPS)])
      if with_deg:
        pltpu.sync_copy(dacc.at[pl.ds(r0, RPS)], deg_a.at[pl.ds(r0, RPS)])

    @pl.when(cid == 1)
    def _():
      pltpu.sync_copy(acc.at[pl.ds(r0, RPS)], agg_b.at[pl.ds(r0, RPS)])
      if with_deg:
        pltpu.sync_copy(dacc.at[pl.ds(r0, RPS)], deg_b.at[pl.ds(r0, RPS)])

  return pl.kernel(body, out_type=tuple(out_type), mesh=mesh,
                   scratch_types=scratch)


_sc_agg_deg = _make_sc_agg(D, True)
_sc_agg_128 = _make_sc_agg(D, False)
_sc_agg_64 = _make_sc_agg(64, False)


def _mm_body(x_ref, w_ref, o_ref):
  o_ref[...] = jnp.dot(x_ref[...], w_ref[...],
                       preferred_element_type=jnp.float32)


def _mm(x, w):
  d_in, d_out = w.shape
  return pl.pallas_call(
      _mm_body,
      grid=(NBLK,),
      in_specs=[pl.BlockSpec((R, d_in), lambda i: (i, 0)),
                pl.BlockSpec((d_in, d_out), lambda i: (0, 0))],
      out_specs=pl.BlockSpec((R, d_out), lambda i: (i, 0)),
      out_shape=jax.ShapeDtypeStruct((N, d_out), jnp.float32),
  )(x, w)


def _upd_body(h_ref, aa_ref, ab_ref, da_ref, db_ref, ws_ref, b_ref, wn_ref,
              hn_ref, pn_ref):
  h = h_ref[...]
  agg = aa_ref[...] + ab_ref[...]
  recip = 1.0 / jnp.maximum(da_ref[...] + db_ref[...], 1.0)
  hn = jnp.maximum(
      jnp.dot(h, ws_ref[...], preferred_element_type=jnp.float32)
      + agg * recip + b_ref[...], 0.0)
  hn_ref[...] = hn
  pn_ref[...] = jnp.dot(hn, wn_ref[...], preferred_element_type=jnp.float32)


def _upd(h, agg_a, agg_b, deg_a, deg_b, w_self, b, w_next):
  d_out = w_self.shape[1]
  d_next = w_next.shape[1]
  return pl.pallas_call(
      _upd_body,
      grid=(NBLK,),
      in_specs=[
          pl.BlockSpec((R, D), lambda i: (i, 0)),
          pl.BlockSpec((R, d_out), lambda i: (i, 0)),
          pl.BlockSpec((R, d_out), lambda i: (i, 0)),
          pl.BlockSpec((R, 1), lambda i: (i, 0)),
          pl.BlockSpec((R, 1), lambda i: (i, 0)),
          pl.BlockSpec((D, d_out), lambda i: (0, 0)),
          pl.BlockSpec((1, d_out), lambda i: (0, 0)),
          pl.BlockSpec((d_out, d_next), lambda i: (0, 0)),
      ],
      out_specs=[pl.BlockSpec((R, d_out), lambda i: (i, 0)),
                 pl.BlockSpec((R, d_next), lambda i: (i, 0))],
      out_shape=[jax.ShapeDtypeStruct((N, d_out), jnp.float32),
                 jax.ShapeDtypeStruct((N, d_next), jnp.float32)],
  )(h, agg_a, agg_b, deg_a, deg_b, w_self, b.reshape(1, d_out), w_next)


def _fin_body(h_ref, aa_ref, ab_ref, da_ref, db_ref, ws_ref, b_ref, o_ref):
  h = h_ref[...]
  agg = aa_ref[...] + ab_ref[...]
  recip = 1.0 / jnp.maximum(da_ref[...] + db_ref[...], 1.0)
  o_ref[...] = (jnp.dot(h, ws_ref[...], preferred_element_type=jnp.float32)
                + agg * recip + b_ref[...])


def _fin(h, agg_a, agg_b, deg_a, deg_b, w_self, b):
  d_out = w_self.shape[1]
  return pl.pallas_call(
      _fin_body,
      grid=(NBLK,),
      in_specs=[
          pl.BlockSpec((R, D), lambda i: (i, 0)),
          pl.BlockSpec((R, d_out), lambda i: (i, 0)),
          pl.BlockSpec((R, d_out), lambda i: (i, 0)),
          pl.BlockSpec((R, 1), lambda i: (i, 0)),
          pl.BlockSpec((R, 1), lambda i: (i, 0)),
          pl.BlockSpec((D, d_out), lambda i: (0, 0)),
          pl.BlockSpec((1, d_out), lambda i: (0, 0)),
      ],
      out_specs=pl.BlockSpec((R, d_out), lambda i: (i, 0)),
      out_shape=jax.ShapeDtypeStruct((N, d_out), jnp.float32),
  )(h, agg_a, agg_b, deg_a, deg_b, w_self, b.reshape(1, d_out))


def kernel(inputs, edge_index, W_self_0, W_neigh_0, b_0, W_self_1, W_neigh_1,
           b_1, W_self_2, W_neigh_2, b_2):
  src = edge_index[0]
  dst = edge_index[1]
  pad = EP - E
  srcp = jnp.concatenate([src, jnp.zeros((pad,), jnp.int32)]).reshape(
      NW * CPW, C)
  dstp = jnp.concatenate([dst, jnp.full((pad,), N, jnp.int32)]).reshape(
      NW * CPW, C)
  zrow = jnp.zeros((RPS, D), jnp.float32)
  zrow64 = jnp.zeros((RPS, 64), jnp.float32)
  zone = jnp.zeros((RPS, 1), jnp.float32)
  onec = jnp.ones((C, 1), jnp.float32)

  p0 = _mm(inputs, W_neigh_0)
  agg_a0, agg_b0, deg_a, deg_b = _sc_agg_deg(p0, srcp, dstp, zrow, zone, onec)
  h1, p1 = _upd(inputs, agg_a0, agg_b0, deg_a, deg_b, W_self_0, b_0, W_neigh_1)
  agg_a1, agg_b1 = _sc_agg_128(p1, srcp, dstp, zrow, zone, onec)
  h2, p2 = _upd(h1, agg_a1, agg_b1, deg_a, deg_b, W_self_1, b_1, W_neigh_2)
  agg_a2, agg_b2 = _sc_agg_64(p2, srcp, dstp, zrow64, zone, onec)
  return _fin(h2, agg_a2, agg_b2, deg_a, deg_b, W_self_2, b_2)


# speedup vs baseline: 2.8955x; 2.8955x over previous
import functools

import jax
import jax.numpy as jnp
from jax import lax
from jax.experimental import pallas as pl
from jax.experimental.pallas import tpu as pltpu
from jax.experimental.pallas import tpu_sc as plsc

N = 10000
D = 128
E = 320000

NC = 2
NS = 16
NW = NC * NS

C = 128
CPW = 80
EP = NW * CPW * C
NP = 10240
RPS = NP // NS

R = 1000
NBLK = N // R

_mesh = plsc.VectorSubcoreMesh(
    core_axis_name="c", subcore_axis_name="s", num_cores=NC, num_subcores=NS)

_AGG_OUT = (jax.ShapeDtypeStruct((NP, D), jnp.float32),
            jax.ShapeDtypeStruct((NP, D), jnp.float32))


def _sc_agg_body(p_hbm, src_hbm, dst_hbm, zrow_hbm, agg_a, agg_b, acc, srcv,
                 dstv, rows, sem):
  cid = lax.axis_index("c")
  sid = lax.axis_index("s")
  wid = sid * NC + cid
  r0 = sid * RPS

  pltpu.sync_copy(zrow_hbm, acc.at[pl.ds(r0, RPS)])
  plsc.subcore_barrier()

  def group(g, carry):
    base = pl.multiple_of(wid * CPW + g * 8, 8)
    pltpu.sync_copy(src_hbm.at[pl.ds(base, 8)], srcv)
    pltpu.sync_copy(dst_hbm.at[pl.ds(base, 8)], dstv)

    def step(j, c2):
      pltpu.async_copy(p_hbm.at[srcv.at[j]], rows, sem).wait()
      pltpu.sync_copy(rows, acc.at[dstv.at[j]], add=True)
      return c2

    lax.fori_loop(0, 8, step, 0)
    return carry

  lax.fori_loop(0, CPW // 8, group, 0)
  plsc.subcore_barrier()

  @pl.when(cid == 0)
  def _():
    pltpu.sync_copy(acc.at[pl.ds(r0, RPS)], agg_a.at[pl.ds(r0, RPS)])

  @pl.when(cid == 1)
  def _():
    pltpu.sync_copy(acc.at[pl.ds(r0, RPS)], agg_b.at[pl.ds(r0, RPS)])


_sc_agg = pl.kernel(
    _sc_agg_body, out_type=_AGG_OUT, mesh=_mesh,
    scratch_types=[
        pltpu.VMEM_SHARED((NP, D), jnp.float32),
        pltpu.VMEM((8, C), jnp.int32),
        pltpu.VMEM((8, C), jnp.int32),
        pltpu.VMEM((C, D), jnp.float32),
        pltpu.SemaphoreType.DMA,
    ])


def _sc_deg_body(dst_hbm, zrow_hbm, one_hbm, deg_a, deg_b, acc, dstv, onesv):
  cid = lax.axis_index("c")
  sid = lax.axis_index("s")
  wid = sid * NC + cid
  r0 = sid * RPS

  pltpu.sync_copy(zrow_hbm, acc.at[pl.ds(r0, RPS)])
  pltpu.sync_copy(one_hbm, onesv)
  plsc.subcore_barrier()

  def group(g, carry):
    base = pl.multiple_of(wid * CPW + g * 8, 8)
    pltpu.sync_copy(dst_hbm.at[pl.ds(base, 8)], dstv)

    def step(j, c2):
      pltpu.sync_copy(onesv, acc.at[dstv.at[j]], add=True)
      return c2

    lax.fori_loop(0, 8, step, 0)
    return carry

  lax.fori_loop(0, CPW // 8, group, 0)
  plsc.subcore_barrier()

  @pl.when(cid == 0)
  def _():
    pltpu.sync_copy(acc.at[pl.ds(r0, RPS)], deg_a.at[pl.ds(r0, RPS)])

  @pl.when(cid == 1)
  def _():
    pltpu.sync_copy(acc.at[pl.ds(r0, RPS)], deg_b.at[pl.ds(r0, RPS)])


_sc_deg = pl.kernel(
    _sc_deg_body, out_type=_AGG_OUT, mesh=_mesh,
    scratch_types=[
        pltpu.VMEM_SHARED((NP, D), jnp.float32),
        pltpu.VMEM((8, C), jnp.int32),
        pltpu.VMEM((C, D), jnp.float32),
    ])


def _layer_body(h_ref, aa_ref, ab_ref, da_ref, db_ref, ws_ref, wn_ref, b_ref,
                o_ref, *, relu):
  h = h_ref[...]
  recip = 1.0 / jnp.maximum(da_ref[:, 0:1] + db_ref[:, 0:1], 1.0)
  h_neigh = (aa_ref[...] + ab_ref[...]) * recip
  o = (jnp.dot(h, ws_ref[...], preferred_element_type=jnp.float32)
       + jnp.dot(h_neigh, wn_ref[...], preferred_element_type=jnp.float32)
       + b_ref[...])
  o_ref[...] = jnp.maximum(o, 0.0) if relu else o


def _layer(h, agg_a, agg_b, deg_a, deg_b, w_self, w_neigh, b, relu):
  d_out = w_self.shape[1]
  return pl.pallas_call(
      functools.partial(_layer_body, relu=relu),
      grid=(NBLK,),
      in_specs=[
          pl.BlockSpec((R, D), lambda i: (i, 0)),
          pl.BlockSpec((R, D), lambda i: (i, 0)),
          pl.BlockSpec((R, D), lambda i: (i, 0)),
          pl.BlockSpec((R, D), lambda i: (i, 0)),
          pl.BlockSpec((R, D), lambda i: (i, 0)),
          pl.BlockSpec((D, d_out), lambda i: (0, 0)),
          pl.BlockSpec((D, d_out), lambda i: (0, 0)),
          pl.BlockSpec((1, d_out), lambda i: (0, 0)),
      ],
      out_specs=pl.BlockSpec((R, d_out), lambda i: (i, 0)),
      out_shape=jax.ShapeDtypeStruct((N, d_out), jnp.float32),
  )(h, agg_a, agg_b, deg_a, deg_b, w_self, w_neigh, b.reshape(1, d_out))


def kernel(inputs, edge_index, W_self_0, W_neigh_0, b_0, W_self_1, W_neigh_1,
           b_1, W_self_2, W_neigh_2, b_2):
  src = edge_index[0]
  dst = edge_index[1]
  pad = EP - E
  srcp = jnp.concatenate([src, jnp.zeros((pad,), jnp.int32)]).reshape(
      NW * CPW, C)
  dstp = jnp.concatenate([dst, jnp.full((pad,), N, jnp.int32)]).reshape(
      NW * CPW, C)
  zrow = jnp.zeros((RPS, D), jnp.float32)
  onec = jnp.ones((C, D), jnp.float32)

  deg_a, deg_b = _sc_deg(dstp, zrow, onec)
  agg_a0, agg_b0 = _sc_agg(inputs, srcp, dstp, zrow)
  h1 = _layer(inputs, agg_a0, agg_b0, deg_a, deg_b, W_self_0, W_neigh_0, b_0,
              True)
  agg_a1, agg_b1 = _sc_agg(h1, srcp, dstp, zrow)
  h2 = _layer(h1, agg_a1, agg_b1, deg_a, deg_b, W_self_1, W_neigh_1, b_1, True)
  agg_a2, agg_b2 = _sc_agg(h2, srcp, dstp, zrow)
  return _layer(h2, agg_a2, agg_b2, deg_a, deg_b, W_self_2, W_neigh_2, b_2,
                False)


# speedup vs baseline: 3.1002x; 1.0707x over previous
import functools

import jax
import jax.numpy as jnp
from jax import lax
from jax.experimental import pallas as pl
from jax.experimental.pallas import tpu as pltpu
from jax.experimental.pallas import tpu_sc as plsc

N = 10000
D = 128
E = 320000

NC = 2
NS = 16
NW = NC * NS

C = 128
CPW = 80
EP = NW * CPW * C
NP = 10240
RPS = NP // NS

R = 1000
NBLK = N // R

_mesh = plsc.VectorSubcoreMesh(
    core_axis_name="c", subcore_axis_name="s", num_cores=NC, num_subcores=NS)

_AGG_OUT = (jax.ShapeDtypeStruct((NP, D), jnp.float32),
            jax.ShapeDtypeStruct((NP, D), jnp.float32))


def _sc_agg_body(p_hbm, src_hbm, dst_hbm, zrow_hbm, agg_a, agg_b, acc, srcv,
                 dstv, rows0, rows1, gsem, ssem):
  cid = lax.axis_index("c")
  sid = lax.axis_index("s")
  wid = sid * NC + cid
  r0 = sid * RPS

  pltpu.sync_copy(zrow_hbm, acc.at[pl.ds(r0, RPS)])
  plsc.subcore_barrier()

  rows = (rows0, rows1)

  def group(g, carry):
    base = pl.multiple_of(wid * CPW + g * 8, 8)
    pltpu.sync_copy(src_hbm.at[pl.ds(base, 8)], srcv)
    pltpu.sync_copy(dst_hbm.at[pl.ds(base, 8)], dstv)

    pltpu.make_async_copy(p_hbm.at[srcv.at[0]], rows[0], gsem).start()
    for j in range(8):
      buf = rows[j % 2]
      nbuf = rows[(j + 1) % 2]
      pltpu.make_async_copy(p_hbm.at[srcv.at[j]], buf, gsem).wait()
      if j + 1 < 8:
        if j >= 1:
          pltpu.make_async_copy(nbuf, acc.at[dstv.at[j - 1]], ssem).wait()
        pltpu.make_async_copy(p_hbm.at[srcv.at[j + 1]], nbuf, gsem).start()
      pltpu.make_async_copy(buf, acc.at[dstv.at[j]], ssem).start(add=True)
    pltpu.make_async_copy(rows[0], acc.at[dstv.at[6]], ssem).wait()
    pltpu.make_async_copy(rows[1], acc.at[dstv.at[7]], ssem).wait()
    return carry

  lax.fori_loop(0, CPW // 8, group, 0)
  plsc.subcore_barrier()

  @pl.when(cid == 0)
  def _():
    pltpu.sync_copy(acc.at[pl.ds(r0, RPS)], agg_a.at[pl.ds(r0, RPS)])

  @pl.when(cid == 1)
  def _():
    pltpu.sync_copy(acc.at[pl.ds(r0, RPS)], agg_b.at[pl.ds(r0, RPS)])


_sc_agg = pl.kernel(
    _sc_agg_body, out_type=_AGG_OUT, mesh=_mesh,
    scratch_types=[
        pltpu.VMEM_SHARED((NP, D), jnp.float32),
        pltpu.VMEM((8, C), jnp.int32),
        pltpu.VMEM((8, C), jnp.int32),
        pltpu.VMEM((C, D), jnp.float32),
        pltpu.VMEM((C, D), jnp.float32),
        pltpu.SemaphoreType.DMA,
        pltpu.SemaphoreType.DMA,
    ])


def _sc_deg_body(dst_hbm, zrow_hbm, one_hbm, deg_a, deg_b, acc, dstv, onesv,
                 ssem):
  cid = lax.axis_index("c")
  sid = lax.axis_index("s")
  wid = sid * NC + cid
  r0 = sid * RPS

  pltpu.sync_copy(zrow_hbm, acc.at[pl.ds(r0, RPS)])
  pltpu.sync_copy(one_hbm, onesv)
  plsc.subcore_barrier()

  def group(g, carry):
    base = pl.multiple_of(wid * CPW + g * 8, 8)
    pltpu.sync_copy(dst_hbm.at[pl.ds(base, 8)], dstv)

    for j in range(8):
      pltpu.make_async_copy(onesv, acc.at[dstv.at[j]], ssem).start(add=True)
    for j in range(8):
      pltpu.make_async_copy(onesv, acc.at[dstv.at[j]], ssem).wait()
    return carry

  lax.fori_loop(0, CPW // 8, group, 0)
  plsc.subcore_barrier()

  @pl.when(cid == 0)
  def _():
    pltpu.sync_copy(acc.at[pl.ds(r0, RPS)], deg_a.at[pl.ds(r0, RPS)])

  @pl.when(cid == 1)
  def _():
    pltpu.sync_copy(acc.at[pl.ds(r0, RPS)], deg_b.at[pl.ds(r0, RPS)])


_sc_deg = pl.kernel(
    _sc_deg_body, out_type=_AGG_OUT, mesh=_mesh,
    scratch_types=[
        pltpu.VMEM_SHARED((NP, D), jnp.float32),
        pltpu.VMEM((8, C), jnp.int32),
        pltpu.VMEM((C, D), jnp.float32),
        pltpu.SemaphoreType.DMA,
    ])


def _layer_body(h_ref, aa_ref, ab_ref, da_ref, db_ref, ws_ref, wn_ref, b_ref,
                o_ref, *, relu):
  h = h_ref[...]
  recip = 1.0 / jnp.maximum(da_ref[:, 0:1] + db_ref[:, 0:1], 1.0)
  h_neigh = (aa_ref[...] + ab_ref[...]) * recip
  o = (jnp.dot(h, ws_ref[...], preferred_element_type=jnp.float32)
       + jnp.dot(h_neigh, wn_ref[...], preferred_element_type=jnp.float32)
       + b_ref[...])
  o_ref[...] = jnp.maximum(o, 0.0) if relu else o


def _layer(h, agg_a, agg_b, deg_a, deg_b, w_self, w_neigh, b, relu):
  d_out = w_self.shape[1]
  return pl.pallas_call(
      functools.partial(_layer_body, relu=relu),
      grid=(NBLK,),
      in_specs=[
          pl.BlockSpec((R, D), lambda i: (i, 0)),
          pl.BlockSpec((R, D), lambda i: (i, 0)),
          pl.BlockSpec((R, D), lambda i: (i, 0)),
          pl.BlockSpec((R, D), lambda i: (i, 0)),
          pl.BlockSpec((R, D), lambda i: (i, 0)),
          pl.BlockSpec((D, d_out), lambda i: (0, 0)),
          pl.BlockSpec((D, d_out), lambda i: (0, 0)),
          pl.BlockSpec((1, d_out), lambda i: (0, 0)),
      ],
      out_specs=pl.BlockSpec((R, d_out), lambda i: (i, 0)),
      out_shape=jax.ShapeDtypeStruct((N, d_out), jnp.float32),
  )(h, agg_a, agg_b, deg_a, deg_b, w_self, w_neigh, b.reshape(1, d_out))


def kernel(inputs, edge_index, W_self_0, W_neigh_0, b_0, W_self_1, W_neigh_1,
           b_1, W_self_2, W_neigh_2, b_2):
  src = edge_index[0]
  dst = edge_index[1]
  pad = EP - E
  srcp = jnp.concatenate([src, jnp.zeros((pad,), jnp.int32)]).reshape(
      NW * CPW, C)
  dstp = jnp.concatenate([dst, jnp.full((pad,), N, jnp.int32)]).reshape(
      NW * CPW, C)
  zrow = jnp.zeros((RPS, D), jnp.float32)
  onec = jnp.ones((C, D), jnp.float32)

  deg_a, deg_b = _sc_deg(dstp, zrow, onec)
  agg_a0, agg_b0 = _sc_agg(inputs, srcp, dstp, zrow)
  h1 = _layer(inputs, agg_a0, agg_b0, deg_a, deg_b, W_self_0, W_neigh_0, b_0,
              True)
  agg_a1, agg_b1 = _sc_agg(h1, srcp, dstp, zrow)
  h2 = _layer(h1, agg_a1, agg_b1, deg_a, deg_b, W_self_1, W_neigh_1, b_1, True)
  agg_a2, agg_b2 = _sc_agg(h2, srcp, dstp, zrow)
  return _layer(h2, agg_a2, agg_b2, deg_a, deg_b, W_self_2, W_neigh_2, b_2,
                False)


# speedup vs baseline: 3.2106x; 1.0356x over previous
import functools

import jax
import jax.numpy as jnp
from jax import lax
from jax.experimental import pallas as pl
from jax.experimental.pallas import tpu as pltpu
from jax.experimental.pallas import tpu_sc as plsc

N = 10000
D = 128
E = 320000

NC = 2
NS = 16
NW = NC * NS

C = 64
CPW = 160
G = 32
RB = 4
EP = NW * CPW * C
NP = 10240
RPS = NP // NS

R = 1000
NBLK = N // R

_mesh = plsc.VectorSubcoreMesh(
    core_axis_name="c", subcore_axis_name="s", num_cores=NC, num_subcores=NS)

_AGG_OUT = (jax.ShapeDtypeStruct((NP, D), jnp.float32),
            jax.ShapeDtypeStruct((NP, D), jnp.float32))


def _sc_agg_body(p_hbm, src_hbm, dst_hbm, zrow_hbm, agg_a, agg_b, acc, srcv,
                 dstv, rows0, rows1, rows2, rows3, gsem, ssem):
  cid = lax.axis_index("c")
  sid = lax.axis_index("s")
  wid = sid * NC + cid
  r0 = sid * RPS

  pltpu.sync_copy(zrow_hbm, acc.at[pl.ds(r0, RPS)])
  plsc.subcore_barrier()

  rows = (rows0, rows1, rows2, rows3)

  def group(g, carry):
    base = pl.multiple_of(wid * CPW + g * G, 8)
    pltpu.sync_copy(src_hbm.at[pl.ds(base, G)], srcv)
    pltpu.sync_copy(dst_hbm.at[pl.ds(base, G)], dstv)

    def gwait(b):
      pltpu.make_async_copy(p_hbm.at[srcv.at[0]], rows[b], gsem).wait()

    def swait(b):
      pltpu.make_async_copy(rows[b], acc.at[dstv.at[0]], ssem).wait()

    for j in range(G):
      if j >= RB:
        swait(j % RB)
      pltpu.make_async_copy(p_hbm.at[srcv.at[j]], rows[j % RB], gsem).start()
      if j >= RB - 1:
        k = j - (RB - 1)
        gwait(k % RB)
        pltpu.make_async_copy(rows[k % RB], acc.at[dstv.at[k]],
                              ssem).start(add=True)
    for k in range(G - (RB - 1), G):
      gwait(k % RB)
      pltpu.make_async_copy(rows[k % RB], acc.at[dstv.at[k]],
                            ssem).start(add=True)
    for k in range(G - RB, G):
      swait(k % RB)
    return carry

  lax.fori_loop(0, CPW // G, group, 0)
  plsc.subcore_barrier()

  @pl.when(cid == 0)
  def _():
    pltpu.sync_copy(acc.at[pl.ds(r0, RPS)], agg_a.at[pl.ds(r0, RPS)])

  @pl.when(cid == 1)
  def _():
    pltpu.sync_copy(acc.at[pl.ds(r0, RPS)], agg_b.at[pl.ds(r0, RPS)])


_sc_agg = pl.kernel(
    _sc_agg_body, out_type=_AGG_OUT, mesh=_mesh,
    scratch_types=[
        pltpu.VMEM_SHARED((NP, D), jnp.float32),
        pltpu.VMEM((G, C), jnp.int32),
        pltpu.VMEM((G, C), jnp.int32),
        pltpu.VMEM((C, D), jnp.float32),
        pltpu.VMEM((C, D), jnp.float32),
        pltpu.VMEM((C, D), jnp.float32),
        pltpu.VMEM((C, D), jnp.float32),
        pltpu.SemaphoreType.DMA,
        pltpu.SemaphoreType.DMA,
    ])


def _sc_deg_body(dst_hbm, zrow_hbm, one_hbm, deg_a, deg_b, acc, dstv, onesv,
                 ssem):
  cid = lax.axis_index("c")
  sid = lax.axis_index("s")
  wid = sid * NC + cid
  r0 = sid * RPS

  pltpu.sync_copy(zrow_hbm, acc.at[pl.ds(r0, RPS)])
  pltpu.sync_copy(one_hbm, onesv)
  plsc.subcore_barrier()

  def group(g, carry):
    base = pl.multiple_of(wid * CPW + g * G, 8)
    pltpu.sync_copy(dst_hbm.at[pl.ds(base, G)], dstv)

    for j in range(G):
      pltpu.make_async_copy(onesv, acc.at[dstv.at[j]], ssem).start(add=True)
    for j in range(G):
      pltpu.make_async_copy(onesv, acc.at[dstv.at[j]], ssem).wait()
    return carry

  lax.fori_loop(0, CPW // G, group, 0)
  plsc.subcore_barrier()

  @pl.when(cid == 0)
  def _():
    pltpu.sync_copy(acc.at[pl.ds(r0, RPS)], deg_a.at[pl.ds(r0, RPS)])

  @pl.when(cid == 1)
  def _():
    pltpu.sync_copy(acc.at[pl.ds(r0, RPS)], deg_b.at[pl.ds(r0, RPS)])


_sc_deg = pl.kernel(
    _sc_deg_body, out_type=_AGG_OUT, mesh=_mesh,
    scratch_types=[
        pltpu.VMEM_SHARED((NP, D), jnp.float32),
        pltpu.VMEM((G, C), jnp.int32),
        pltpu.VMEM((C, D), jnp.float32),
        pltpu.SemaphoreType.DMA,
    ])


def _layer_body(h_ref, aa_ref, ab_ref, da_ref, db_ref, ws_ref, wn_ref, b_ref,
                o_ref, *, relu):
  h = h_ref[...]
  recip = 1.0 / jnp.maximum(da_ref[:, 0:1] + db_ref[:, 0:1], 1.0)
  h_neigh = (aa_ref[...] + ab_ref[...]) * recip
  o = (jnp.dot(h, ws_ref[...], preferred_element_type=jnp.float32)
       + jnp.dot(h_neigh, wn_ref[...], preferred_element_type=jnp.float32)
       + b_ref[...])
  o_ref[...] = jnp.maximum(o, 0.0) if relu else o


def _layer(h, agg_a, agg_b, deg_a, deg_b, w_self, w_neigh, b, relu):
  d_out = w_self.shape[1]
  return pl.pallas_call(
      functools.partial(_layer_body, relu=relu),
      grid=(NBLK,),
      in_specs=[
          pl.BlockSpec((R, D), lambda i: (i, 0)),
          pl.BlockSpec((R, D), lambda i: (i, 0)),
          pl.BlockSpec((R, D), lambda i: (i, 0)),
          pl.BlockSpec((R, D), lambda i: (i, 0)),
          pl.BlockSpec((R, D), lambda i: (i, 0)),
          pl.BlockSpec((D, d_out), lambda i: (0, 0)),
          pl.BlockSpec((D, d_out), lambda i: (0, 0)),
          pl.BlockSpec((1, d_out), lambda i: (0, 0)),
      ],
      out_specs=pl.BlockSpec((R, d_out), lambda i: (i, 0)),
      out_shape=jax.ShapeDtypeStruct((N, d_out), jnp.float32),
  )(h, agg_a, agg_b, deg_a, deg_b, w_self, w_neigh, b.reshape(1, d_out))


def kernel(inputs, edge_index, W_self_0, W_neigh_0, b_0, W_self_1, W_neigh_1,
           b_1, W_self_2, W_neigh_2, b_2):
  src = edge_index[0]
  dst = edge_index[1]
  pad = EP - E
  srcp = jnp.concatenate([src, jnp.zeros((pad,), jnp.int32)]).reshape(
      NW * CPW, C)
  dstp = jnp.concatenate([dst, jnp.full((pad,), N, jnp.int32)]).reshape(
      NW * CPW, C)
  zrow = jnp.zeros((RPS, D), jnp.float32)
  onec = jnp.ones((C, D), jnp.float32)

  deg_a, deg_b = _sc_deg(dstp, zrow, onec)
  agg_a0, agg_b0 = _sc_agg(inputs, srcp, dstp, zrow)
  h1 = _layer(inputs, agg_a0, agg_b0, deg_a, deg_b, W_self_0, W_neigh_0, b_0,
              True)
  agg_a1, agg_b1 = _sc_agg(h1, srcp, dstp, zrow)
  h2 = _layer(h1, agg_a1, agg_b1, deg_a, deg_b, W_self_1, W_neigh_1, b_1, True)
  agg_a2, agg_b2 = _sc_agg(h2, srcp, dstp, zrow)
  return _layer(h2, agg_a2, agg_b2, deg_a, deg_b, W_self_2, W_neigh_2, b_2,
                False)


# speedup vs baseline: 3.3862x; 1.0547x over previous
import functools

import jax
import jax.numpy as jnp
from jax import lax
from jax.experimental import pallas as pl
from jax.experimental.pallas import tpu as pltpu
from jax.experimental.pallas import tpu_sc as plsc

N = 10000
D = 128
E = 320000

NC = 2
NS = 16
NW = NC * NS

C = 32
CPW = 320
G = 32
RB = 8
EP = NW * CPW * C
NP = 10240
RPS = NP // NS

R = 1000
NBLK = N // R

_mesh = plsc.VectorSubcoreMesh(
    core_axis_name="c", subcore_axis_name="s", num_cores=NC, num_subcores=NS)

_AGG_OUT = (jax.ShapeDtypeStruct((NP, D), jnp.float32),
            jax.ShapeDtypeStruct((NP, D), jnp.float32))


def _sc_agg_body(p_hbm, src_hbm, dst_hbm, zrow_hbm, agg_a, agg_b, acc, srcv,
                 dstv, rows0, rows1, rows2, rows3, rows4, rows5, rows6, rows7,
                 gsem, ssem):
  cid = lax.axis_index("c")
  sid = lax.axis_index("s")
  wid = sid * NC + cid
  r0 = sid * RPS

  pltpu.sync_copy(zrow_hbm, acc.at[pl.ds(r0, RPS)])
  plsc.subcore_barrier()

  rows = (rows0, rows1, rows2, rows3, rows4, rows5, rows6, rows7)

  def group(g, carry):
    base = pl.multiple_of(wid * CPW + g * G, 8)
    pltpu.sync_copy(src_hbm.at[pl.ds(base, G)], srcv)
    pltpu.sync_copy(dst_hbm.at[pl.ds(base, G)], dstv)

    def gwait(b):
      pltpu.make_async_copy(p_hbm.at[srcv.at[0]], rows[b], gsem).wait()

    def swait(b):
      pltpu.make_async_copy(rows[b], acc.at[dstv.at[0]], ssem).wait()

    for j in range(G):
      if j >= RB:
        swait(j % RB)
      pltpu.make_async_copy(p_hbm.at[srcv.at[j]], rows[j % RB], gsem).start()
      if j >= RB - 1:
        k = j - (RB - 1)
        gwait(k % RB)
        pltpu.make_async_copy(rows[k % RB], acc.at[dstv.at[k]],
                              ssem).start(add=True)
    for k in range(G - (RB - 1), G):
      gwait(k % RB)
      pltpu.make_async_copy(rows[k % RB], acc.at[dstv.at[k]],
                            ssem).start(add=True)
    for k in range(G - RB, G):
      swait(k % RB)
    return carry

  lax.fori_loop(0, CPW // G, group, 0)
  plsc.subcore_barrier()

  @pl.when(cid == 0)
  def _():
    pltpu.sync_copy(acc.at[pl.ds(r0, RPS)], agg_a.at[pl.ds(r0, RPS)])

  @pl.when(cid == 1)
  def _():
    pltpu.sync_copy(acc.at[pl.ds(r0, RPS)], agg_b.at[pl.ds(r0, RPS)])


_sc_agg = pl.kernel(
    _sc_agg_body, out_type=_AGG_OUT, mesh=_mesh,
    scratch_types=[
        pltpu.VMEM_SHARED((NP, D), jnp.float32),
        pltpu.VMEM((G, C), jnp.int32),
        pltpu.VMEM((G, C), jnp.int32),
        pltpu.VMEM((C, D), jnp.float32),
        pltpu.VMEM((C, D), jnp.float32),
        pltpu.VMEM((C, D), jnp.float32),
        pltpu.VMEM((C, D), jnp.float32),
        pltpu.VMEM((C, D), jnp.float32),
        pltpu.VMEM((C, D), jnp.float32),
        pltpu.VMEM((C, D), jnp.float32),
        pltpu.VMEM((C, D), jnp.float32),
        pltpu.SemaphoreType.DMA,
        pltpu.SemaphoreType.DMA,
    ])


def _sc_deg_body(dst_hbm, zrow_hbm, one_hbm, deg_a, deg_b, acc, dstv, onesv,
                 ssem):
  cid = lax.axis_index("c")
  sid = lax.axis_index("s")
  wid = sid * NC + cid
  r0 = sid * RPS

  pltpu.sync_copy(zrow_hbm, acc.at[pl.ds(r0, RPS)])
  pltpu.sync_copy(one_hbm, onesv)
  plsc.subcore_barrier()

  def group(g, carry):
    base = pl.multiple_of(wid * CPW + g * G, 8)
    pltpu.sync_copy(dst_hbm.at[pl.ds(base, G)], dstv)

    for j in range(G):
      pltpu.make_async_copy(onesv, acc.at[dstv.at[j]], ssem).start(add=True)
    for j in range(G):
      pltpu.make_async_copy(onesv, acc.at[dstv.at[j]], ssem).wait()
    return carry

  lax.fori_loop(0, CPW // G, group, 0)
  plsc.subcore_barrier()

  @pl.when(cid == 0)
  def _():
    pltpu.sync_copy(acc.at[pl.ds(r0, RPS)], deg_a.at[pl.ds(r0, RPS)])

  @pl.when(cid == 1)
  def _():
    pltpu.sync_copy(acc.at[pl.ds(r0, RPS)], deg_b.at[pl.ds(r0, RPS)])


_sc_deg = pl.kernel(
    _sc_deg_body, out_type=_AGG_OUT, mesh=_mesh,
    scratch_types=[
        pltpu.VMEM_SHARED((NP, D), jnp.float32),
        pltpu.VMEM((G, C), jnp.int32),
        pltpu.VMEM((C, D), jnp.float32),
        pltpu.SemaphoreType.DMA,
    ])


def _layer_body(h_ref, aa_ref, ab_ref, da_ref, db_ref, ws_ref, wn_ref, b_ref,
                o_ref, *, relu):
  h = h_ref[...]
  recip = 1.0 / jnp.maximum(da_ref[:, 0:1] + db_ref[:, 0:1], 1.0)
  h_neigh = (aa_ref[...] + ab_ref[...]) * recip
  o = (jnp.dot(h, ws_ref[...], preferred_element_type=jnp.float32)
       + jnp.dot(h_neigh, wn_ref[...], preferred_element_type=jnp.float32)
       + b_ref[...])
  o_ref[...] = jnp.maximum(o, 0.0) if relu else o


def _layer(h, agg_a, agg_b, deg_a, deg_b, w_self, w_neigh, b, relu):
  d_out = w_self.shape[1]
  return pl.pallas_call(
      functools.partial(_layer_body, relu=relu),
      grid=(NBLK,),
      in_specs=[
          pl.BlockSpec((R, D), lambda i: (i, 0)),
          pl.BlockSpec((R, D), lambda i: (i, 0)),
          pl.BlockSpec((R, D), lambda i: (i, 0)),
          pl.BlockSpec((R, D), lambda i: (i, 0)),
          pl.BlockSpec((R, D), lambda i: (i, 0)),
          pl.BlockSpec((D, d_out), lambda i: (0, 0)),
          pl.BlockSpec((D, d_out), lambda i: (0, 0)),
          pl.BlockSpec((1, d_out), lambda i: (0, 0)),
      ],
      out_specs=pl.BlockSpec((R, d_out), lambda i: (i, 0)),
      out_shape=jax.ShapeDtypeStruct((N, d_out), jnp.float32),
  )(h, agg_a, agg_b, deg_a, deg_b, w_self, w_neigh, b.reshape(1, d_out))


def kernel(inputs, edge_index, W_self_0, W_neigh_0, b_0, W_self_1, W_neigh_1,
           b_1, W_self_2, W_neigh_2, b_2):
  src = edge_index[0]
  dst = edge_index[1]
  pad = EP - E
  srcp = jnp.concatenate([src, jnp.zeros((pad,), jnp.int32)]).reshape(
      NW * CPW, C)
  dstp = jnp.concatenate([dst, jnp.full((pad,), N, jnp.int32)]).reshape(
      NW * CPW, C)
  zrow = jnp.zeros((RPS, D), jnp.float32)
  onec = jnp.ones((C, D), jnp.float32)

  deg_a, deg_b = _sc_deg(dstp, zrow, onec)
  agg_a0, agg_b0 = _sc_agg(inputs, srcp, dstp, zrow)
  h1 = _layer(inputs, agg_a0, agg_b0, deg_a, deg_b, W_self_0, W_neigh_0, b_0,
              True)
  agg_a1, agg_b1 = _sc_agg(h1, srcp, dstp, zrow)
  h2 = _layer(h1, agg_a1, agg_b1, deg_a, deg_b, W_self_1, W_neigh_1, b_1, True)
  agg_a2, agg_b2 = _sc_agg(h2, srcp, dstp, zrow)
  return _layer(h2, agg_a2, agg_b2, deg_a, deg_b, W_self_2, W_neigh_2, b_2,
                False)


# speedup vs baseline: 3.4212x; 1.0103x over previous
import functools

import jax
import jax.numpy as jnp
from jax import lax
from jax.experimental import pallas as pl
from jax.experimental.pallas import tpu as pltpu
from jax.experimental.pallas import tpu_sc as plsc

N = 10000
D = 128
E = 320000

NC = 2
NS = 16
NW = NC * NS

C = 32
CPW = 320
G = 64
RB = 8
EP = NW * CPW * C
NP = 10240
RPS = NP // NS

R = 1000
NBLK = N // R

_mesh = plsc.VectorSubcoreMesh(
    core_axis_name="c", subcore_axis_name="s", num_cores=NC, num_subcores=NS)

_AGG_OUT = (jax.ShapeDtypeStruct((NP, D), jnp.float32),
            jax.ShapeDtypeStruct((NP, D), jnp.float32))


def _sc_agg_body(p_hbm, src_hbm, dst_hbm, zrow_hbm, agg_a, agg_b, acc, srcv,
                 dstv, rows0, rows1, rows2, rows3, rows4, rows5, rows6, rows7,
                 gsem, ssem):
  cid = lax.axis_index("c")
  sid = lax.axis_index("s")
  wid = sid * NC + cid
  r0 = sid * RPS

  pltpu.sync_copy(zrow_hbm, acc.at[pl.ds(r0, RPS)])
  plsc.subcore_barrier()

  rows = (rows0, rows1, rows2, rows3, rows4, rows5, rows6, rows7)

  def group(g, carry):
    base = pl.multiple_of(wid * CPW + g * G, 8)
    pltpu.sync_copy(src_hbm.at[pl.ds(base, G)], srcv)
    pltpu.sync_copy(dst_hbm.at[pl.ds(base, G)], dstv)

    def gwait(b):
      pltpu.make_async_copy(p_hbm.at[srcv.at[0]], rows[b], gsem).wait()

    def swait(b):
      pltpu.make_async_copy(rows[b], acc.at[dstv.at[0]], ssem).wait()

    for j in range(G):
      if j >= RB:
        swait(j % RB)
      pltpu.make_async_copy(p_hbm.at[srcv.at[j]], rows[j % RB], gsem).start()
      if j >= RB - 1:
        k = j - (RB - 1)
        gwait(k % RB)
        pltpu.make_async_copy(rows[k % RB], acc.at[dstv.at[k]],
                              ssem).start(add=True)
    for k in range(G - (RB - 1), G):
      gwait(k % RB)
      pltpu.make_async_copy(rows[k % RB], acc.at[dstv.at[k]],
                            ssem).start(add=True)
    for k in range(G - RB, G):
      swait(k % RB)
    return carry

  lax.fori_loop(0, CPW // G, group, 0)
  plsc.subcore_barrier()

  @pl.when(cid == 0)
  def _():
    pltpu.sync_copy(acc.at[pl.ds(r0, RPS)], agg_a.at[pl.ds(r0, RPS)])

  @pl.when(cid == 1)
  def _():
    pltpu.sync_copy(acc.at[pl.ds(r0, RPS)], agg_b.at[pl.ds(r0, RPS)])


_sc_agg = pl.kernel(
    _sc_agg_body, out_type=_AGG_OUT, mesh=_mesh,
    scratch_types=[
        pltpu.VMEM_SHARED((NP, D), jnp.float32),
        pltpu.VMEM((G, C), jnp.int32),
        pltpu.VMEM((G, C), jnp.int32),
        pltpu.VMEM((C, D), jnp.float32),
        pltpu.VMEM((C, D), jnp.float32),
        pltpu.VMEM((C, D), jnp.float32),
        pltpu.VMEM((C, D), jnp.float32),
        pltpu.VMEM((C, D), jnp.float32),
        pltpu.VMEM((C, D), jnp.float32),
        pltpu.VMEM((C, D), jnp.float32),
        pltpu.VMEM((C, D), jnp.float32),
        pltpu.SemaphoreType.DMA,
        pltpu.SemaphoreType.DMA,
    ])


def _sc_deg_body(dst_hbm, zrow_hbm, one_hbm, deg_a, deg_b, acc, dstv, onesv,
                 ssem):
  cid = lax.axis_index("c")
  sid = lax.axis_index("s")
  wid = sid * NC + cid
  r0 = sid * RPS

  pltpu.sync_copy(zrow_hbm, acc.at[pl.ds(r0, RPS)])
  pltpu.sync_copy(one_hbm, onesv)
  plsc.subcore_barrier()

  def group(g, carry):
    base = pl.multiple_of(wid * CPW + g * G, 8)
    pltpu.sync_copy(dst_hbm.at[pl.ds(base, G)], dstv)

    for j in range(G):
      pltpu.make_async_copy(onesv, acc.at[dstv.at[j]], ssem).start(add=True)
    for j in range(G):
      pltpu.make_async_copy(onesv, acc.at[dstv.at[j]], ssem).wait()
    return carry

  lax.fori_loop(0, CPW // G, group, 0)
  plsc.subcore_barrier()

  @pl.when(cid == 0)
  def _():
    pltpu.sync_copy(acc.at[pl.ds(r0, RPS)], deg_a.at[pl.ds(r0, RPS)])

  @pl.when(cid == 1)
  def _():
    pltpu.sync_copy(acc.at[pl.ds(r0, RPS)], deg_b.at[pl.ds(r0, RPS)])


_sc_deg = pl.kernel(
    _sc_deg_body, out_type=_AGG_OUT, mesh=_mesh,
    scratch_types=[
        pltpu.VMEM_SHARED((NP, D), jnp.float32),
        pltpu.VMEM((G, C), jnp.int32),
        pltpu.VMEM((C, D), jnp.float32),
        pltpu.SemaphoreType.DMA,
    ])


def _layer_body(h_ref, aa_ref, ab_ref, da_ref, db_ref, ws_ref, wn_ref, b_ref,
                o_ref, *, relu):
  h = h_ref[...]
  recip = 1.0 / jnp.maximum(da_ref[:, 0:1] + db_ref[:, 0:1], 1.0)
  h_neigh = (aa_ref[...] + ab_ref[...]) * recip
  o = (jnp.dot(h, ws_ref[...], preferred_element_type=jnp.float32)
       + jnp.dot(h_neigh, wn_ref[...], preferred_element_type=jnp.float32)
       + b_ref[...])
  o_ref[...] = jnp.maximum(o, 0.0) if relu else o


def _layer(h, agg_a, agg_b, deg_a, deg_b, w_self, w_neigh, b, relu):
  d_out = w_self.shape[1]
  return pl.pallas_call(
      functools.partial(_layer_body, relu=relu),
      grid=(NBLK,),
      in_specs=[
          pl.BlockSpec((R, D), lambda i: (i, 0)),
          pl.BlockSpec((R, D), lambda i: (i, 0)),
          pl.BlockSpec((R, D), lambda i: (i, 0)),
          pl.BlockSpec((R, D), lambda i: (i, 0)),
          pl.BlockSpec((R, D), lambda i: (i, 0)),
          pl.BlockSpec((D, d_out), lambda i: (0, 0)),
          pl.BlockSpec((D, d_out), lambda i: (0, 0)),
          pl.BlockSpec((1, d_out), lambda i: (0, 0)),
      ],
      out_specs=pl.BlockSpec((R, d_out), lambda i: (i, 0)),
      out_shape=jax.ShapeDtypeStruct((N, d_out), jnp.float32),
  )(h, agg_a, agg_b, deg_a, deg_b, w_self, w_neigh, b.reshape(1, d_out))


def kernel(inputs, edge_index, W_self_0, W_neigh_0, b_0, W_self_1, W_neigh_1,
           b_1, W_self_2, W_neigh_2, b_2):
  src = edge_index[0]
  dst = edge_index[1]
  pad = EP - E
  srcp = jnp.concatenate([src, jnp.zeros((pad,), jnp.int32)]).reshape(
      NW * CPW, C)
  dstp = jnp.concatenate([dst, jnp.full((pad,), N, jnp.int32)]).reshape(
      NW * CPW, C)
  zrow = jnp.zeros((RPS, D), jnp.float32)
  onec = jnp.ones((C, D), jnp.float32)

  deg_a, deg_b = _sc_deg(dstp, zrow, onec)
  agg_a0, agg_b0 = _sc_agg(inputs, srcp, dstp, zrow)
  h1 = _layer(inputs, agg_a0, agg_b0, deg_a, deg_b, W_self_0, W_neigh_0, b_0,
              True)
  agg_a1, agg_b1 = _sc_agg(h1, srcp, dstp, zrow)
  h2 = _layer(h1, agg_a1, agg_b1, deg_a, deg_b, W_self_1, W_neigh_1, b_1, True)
  agg_a2, agg_b2 = _sc_agg(h2, srcp, dstp, zrow)
  return _layer(h2, agg_a2, agg_b2, deg_a, deg_b, W_self_2, W_neigh_2, b_2,
                False)
